# baseline (speedup 1.0000x reference)
import functools

import jax
import jax.numpy as jnp
from jax import lax
from jax.experimental import pallas as pl
from jax.experimental.pallas import tpu as pltpu
from jax.experimental.pallas import tpu_sc as plsc

NUM_NODES = 100000
NODE_DIM = 128
MEMORY_DIM = 128
EMBED_DIM = 64
E = 500000

NCHUNK = 2
EC = E // NCHUNK

BN = 2000
BE = 2000
HB = BE // 2

NC = 2
NS = 16
NW = NC * NS
CHUNK = 128
NCH = 125
NPW = CHUNK * NCH
BPAD = NW * NPW
LIDX = BPAD
RING = 4
NLANE = 16


def _proj_body(mem_ref, w_ref, out_ref):
    out_ref[...] = jnp.dot(mem_ref[...], w_ref[...],
                           preferred_element_type=jnp.float32)


def _edge_body(sf_ref, df_ref, gs_ref, gd_ref, w1b_ref, b1_ref, w2_ref,
               b2_ref, out_ref):
    fs = jnp.dot(sf_ref[...], w1b_ref[...], preferred_element_type=jnp.float32)
    fd = jnp.dot(df_ref[...], w1b_ref[...], preferred_element_type=jnp.float32)
    g2s = gs_ref[...]
    g2d = gd_ref[...]
    b1v = b1_ref[...]
    w2 = w2_ref[...]
    b2v = b2_ref[...]

    def emb(h):
        return jnp.dot(h, w2, preferred_element_type=jnp.float32) + b2v

    hs_lo = jnp.maximum(g2s[:, :EMBED_DIM] + fs[:HB] + b1v, 0.0)
    hs_hi = jnp.maximum(g2s[:, EMBED_DIM:] + fs[HB:] + b1v, 0.0)
    hd_lo = jnp.maximum(g2d[:, :EMBED_DIM] + fd[:HB] + b1v, 0.0)
    hd_hi = jnp.maximum(g2d[:, EMBED_DIM:] + fd[HB:] + b1v, 0.0)
    l_lo = jnp.sum(emb(hs_lo) * emb(hd_lo), axis=-1)
    l_hi = jnp.sum(emb(hs_hi) * emb(hd_hi), axis=-1)
    out_ref[...] = jnp.concatenate([l_lo, l_hi]).reshape(1, 1, BE)


@functools.partial(
    pl.kernel,
    out_type=jax.ShapeDtypeStruct((BPAD, EMBED_DIM), jnp.float32),
    mesh=plsc.VectorSubcoreMesh(core_axis_name="c", subcore_axis_name="s"),
    compiler_params=pltpu.CompilerParams(use_tc_tiling_on_sc=False,
                                         needs_layout_passes=False),
    scratch_types=(
        [pltpu.VMEM((NPW,), jnp.int32)]
        + [pltpu.VMEM((CHUNK,), jnp.int32)] * RING
        + [pltpu.VMEM((CHUNK, EMBED_DIM), jnp.float32)] * RING
        + [pltpu.SemaphoreType.DMA] * RING
    ),
)
def _sc_gather(table_hbm, idx_hbm, out_hbm, idx_raw_v, i0, i1, i2, i3,
               r0, r1, r2, r3, s0, s1, s2, s3):
    idxb = (i0, i1, i2, i3)
    rows = (r0, r1, r2, r3)
    sems = (s0, s1, s2, s3)
    wid = lax.axis_index("s") * NC + lax.axis_index("c")
    base = wid * NPW
    pltpu.sync_copy(idx_hbm.at[pl.ds(base, NPW)], idx_raw_v)

    lane = lax.iota(jnp.int32, NLANE)
    parity = lane & 1

    def fill_idx(b, qrel, r):
        for v in range(CHUNK // NLANE):
            part = r + v * NLANE + lane
            over = jnp.where(part >= BE, 1, 0)
            sig = (qrel + over) * BE + parity * HB + ((part - over * BE) >> 1)
            idxb[b][pl.ds(v * NLANE, NLANE)] = plsc.load_gather(
                idx_raw_v, (sig,))

    def chunk_copy(j, b):
        return pltpu.make_async_copy(
            table_hbm.at[idxb[b]], rows[b], sems[b])

    for b in range(RING):
        fill_idx(b, 0, b * CHUNK)
        chunk_copy(b, b).start()

    def body(m, carry):
        qrel, r = carry
        for b in range(RING):
            j = RING * m + b
            chunk_copy(j, b).wait()
            pltpu.sync_copy(rows[b],
                            out_hbm.at[pl.ds(base + j * CHUNK, CHUNK)])
            fill_idx(b, qrel, r + b * CHUNK)
            chunk_copy(RING * m + RING + b, b).start()
        r = r + RING * CHUNK
        wrap = jnp.where(r >= BE, 1, 0)
        return qrel + wrap, r - wrap * BE

    nfull = (NCH - 1) // RING - 1
    lax.fori_loop(0, nfull, body,
                  (jnp.int32((RING * CHUNK) // BE),
                   jnp.int32((RING * CHUNK) % BE)))

    for b in range(RING):
        j = NCH - 1 - RING + b
        chunk_copy(j, b).wait()
        pltpu.sync_copy(rows[b], out_hbm.at[pl.ds(base + j * CHUNK, CHUNK)])
    j = NCH - 1
    fill_idx(0, (j * CHUNK) // BE, (j * CHUNK) % BE)
    chunk_copy(j, 0).start()
    chunk_copy(j, 0).wait()
    pltpu.sync_copy(rows[0], out_hbm.at[pl.ds(base + j * CHUNK, CHUNK)])


def kernel(src_nodes, dst_nodes, src_features, dst_features, memory,
           W1, b1, W2, b2):
    src_nodes = src_nodes.astype(jnp.int32)
    dst_nodes = dst_nodes.astype(jnp.int32)
    w1_mem = W1[:MEMORY_DIM]
    w1_feat = W1[MEMORY_DIM:]
    b1r = b1.reshape(1, EMBED_DIM)
    b2r = b2.reshape(1, EMBED_DIM)

    memory_proj = pl.pallas_call(
        _proj_body,
        grid=(NUM_NODES // BN,),
        in_specs=[
            pl.BlockSpec((BN, MEMORY_DIM), lambda i: (i, 0)),
            pl.BlockSpec((MEMORY_DIM, EMBED_DIM), lambda i: (0, 0)),
        ],
        out_specs=pl.BlockSpec((BN, EMBED_DIM), lambda i: (i, 0)),
        out_shape=jax.ShapeDtypeStruct((NUM_NODES, EMBED_DIM), jnp.float32),
    )(memory, w1_mem)

    pad = jnp.zeros((LIDX - 2 * EC,), jnp.int32)
    nblk = EC // BE
    chunks = []
    for c in range(NCHUNK):
        idx = jnp.concatenate(
            [lax.slice(src_nodes, (c * EC,), ((c + 1) * EC,)),
             lax.slice(dst_nodes, (c * EC,), ((c + 1) * EC,)),
             pad])
        gathered = _sc_gather(memory_proj, idx)
        g2 = gathered.reshape(BPAD // 2, 2 * EMBED_DIM)
        logits_c = pl.pallas_call(
            _edge_body,
            grid=(nblk,),
            in_specs=[
                pl.BlockSpec((BE, NODE_DIM),
                             lambda i, c=c: (i + c * nblk, 0)),
                pl.BlockSpec((BE, NODE_DIM),
                             lambda i, c=c: (i + c * nblk, 0)),
                pl.BlockSpec((HB, 2 * EMBED_DIM), lambda i: (i, 0)),
                pl.BlockSpec((HB, 2 * EMBED_DIM), lambda i: (i + nblk, 0)),
                pl.BlockSpec((NODE_DIM, EMBED_DIM), lambda i: (0, 0)),
                pl.BlockSpec((1, EMBED_DIM), lambda i: (0, 0)),
                pl.BlockSpec((EMBED_DIM, EMBED_DIM), lambda i: (0, 0)),
                pl.BlockSpec((1, EMBED_DIM), lambda i: (0, 0)),
            ],
            out_specs=pl.BlockSpec((1, 1, BE), lambda i: (i, 0, 0)),
            out_shape=jax.ShapeDtypeStruct((nblk, 1, BE), jnp.float32),
        )(src_features, dst_features, g2, g2, w1_feat, b1r, W2, b2r)
        chunks.append(logits_c)
    return jnp.concatenate(chunks).reshape(E)


# speedup vs baseline: 1.3179x; 1.3179x over previous
import functools

import jax
import jax.numpy as jnp
from jax import lax
from jax.experimental import pallas as pl
from jax.experimental.pallas import tpu as pltpu
from jax.experimental.pallas import tpu_sc as plsc

NUM_NODES = 100000
NODE_DIM = 128
MEMORY_DIM = 128
EMBED_DIM = 64
E = 500000

NCHUNK = 2
EC = E // NCHUNK

BN = 2000
BE = 10000
PB = 2000
PH = PB // 2

NC = 2
NS = 16
NW = NC * NS
CHUNK = 128
NCH = 125
NPW = CHUNK * NCH
BPAD = NW * NPW
LIDX = BPAD
RING = 4
NLANE = 16


def _proj_body(mem_ref, w_ref, out_ref):
    out_ref[...] = jnp.dot(mem_ref[...], w_ref[...],
                           preferred_element_type=jnp.float32)


def _unpair(g2):
    parts = []
    for p in range(BE // PB):
        seg = g2[p * PH:(p + 1) * PH]
        parts.append(seg[:, :EMBED_DIM])
        parts.append(seg[:, EMBED_DIM:])
    return jnp.concatenate(parts, axis=0)


def _edge_body(sf_ref, df_ref, gs_ref, gd_ref, w1b_ref, b1_ref, w2_ref,
               b2_ref, out_ref):
    fs = jnp.dot(sf_ref[...], w1b_ref[...], preferred_element_type=jnp.float32)
    fd = jnp.dot(df_ref[...], w1b_ref[...], preferred_element_type=jnp.float32)
    b1v = b1_ref[...]
    w2 = w2_ref[...]
    b2v = b2_ref[...]

    def emb(h):
        return jnp.dot(h, w2, preferred_element_type=jnp.float32) + b2v

    hs = jnp.maximum(_unpair(gs_ref[...]) + fs + b1v, 0.0)
    hd = jnp.maximum(_unpair(gd_ref[...]) + fd + b1v, 0.0)
    out_ref[...] = jnp.sum(emb(hs) * emb(hd), axis=-1).reshape(1, 1, BE)


@functools.partial(
    pl.kernel,
    out_type=jax.ShapeDtypeStruct((BPAD, EMBED_DIM), jnp.float32),
    mesh=plsc.VectorSubcoreMesh(core_axis_name="c", subcore_axis_name="s"),
    compiler_params=pltpu.CompilerParams(use_tc_tiling_on_sc=False,
                                         needs_layout_passes=False),
    scratch_types=(
        [pltpu.VMEM((NPW,), jnp.int32)]
        + [pltpu.VMEM((CHUNK,), jnp.int32)] * RING
        + [pltpu.VMEM((CHUNK, EMBED_DIM), jnp.float32)] * RING
        + [pltpu.SemaphoreType.DMA] * RING
    ),
)
def _sc_gather(table_hbm, idx_hbm, out_hbm, idx_raw_v, i0, i1, i2, i3,
               r0, r1, r2, r3, s0, s1, s2, s3):
    idxb = (i0, i1, i2, i3)
    rows = (r0, r1, r2, r3)
    sems = (s0, s1, s2, s3)
    wid = lax.axis_index("s") * NC + lax.axis_index("c")
    base = wid * NPW
    pltpu.sync_copy(idx_hbm.at[pl.ds(base, NPW)], idx_raw_v)

    lane = lax.iota(jnp.int32, NLANE)
    parity = lane & 1

    def fill_idx(b, qrel, r):
        for v in range(CHUNK // NLANE):
            part = r + v * NLANE + lane
            over = jnp.where(part >= PB, 1, 0)
            sig = (qrel + over) * PB + parity * PH + ((part - over * PB) >> 1)
            idxb[b][pl.ds(v * NLANE, NLANE)] = plsc.load_gather(
                idx_raw_v, (sig,))

    def chunk_copy(j, b):
        return pltpu.make_async_copy(
            table_hbm.at[idxb[b]], rows[b], sems[b])

    for b in range(RING):
        fill_idx(b, 0, b * CHUNK)
        chunk_copy(b, b).start()

    def body(m, carry):
        qrel, r = carry
        for b in range(RING):
            j = RING * m + b
            chunk_copy(j, b).wait()
            pltpu.sync_copy(rows[b],
                            out_hbm.at[pl.ds(base + j * CHUNK, CHUNK)])
            fill_idx(b, qrel, r + b * CHUNK)
            chunk_copy(RING * m + RING + b, b).start()
        r = r + RING * CHUNK
        wrap = jnp.where(r >= PB, 1, 0)
        return qrel + wrap, r - wrap * PB

    nfull = (NCH - 1) // RING - 1
    lax.fori_loop(0, nfull, body,
                  (jnp.int32((RING * CHUNK) // PB),
                   jnp.int32((RING * CHUNK) % PB)))

    for b in range(RING):
        j = NCH - 1 - RING + b
        chunk_copy(j, b).wait()
        pltpu.sync_copy(rows[b], out_hbm.at[pl.ds(base + j * CHUNK, CHUNK)])
    j = NCH - 1
    fill_idx(0, (j * CHUNK) // PB, (j * CHUNK) % PB)
    chunk_copy(j, 0).start()
    chunk_copy(j, 0).wait()
    pltpu.sync_copy(rows[0], out_hbm.at[pl.ds(base + j * CHUNK, CHUNK)])


def kernel(src_nodes, dst_nodes, src_features, dst_features, memory,
           W1, b1, W2, b2):
    src_nodes = src_nodes.astype(jnp.int32)
    dst_nodes = dst_nodes.astype(jnp.int32)
    w1_mem = W1[:MEMORY_DIM]
    w1_feat = W1[MEMORY_DIM:]
    b1r = b1.reshape(1, EMBED_DIM)
    b2r = b2.reshape(1, EMBED_DIM)

    memory_proj = pl.pallas_call(
        _proj_body,
        grid=(NUM_NODES // BN,),
        in_specs=[
            pl.BlockSpec((BN, MEMORY_DIM), lambda i: (i, 0)),
            pl.BlockSpec((MEMORY_DIM, EMBED_DIM), lambda i: (0, 0)),
        ],
        out_specs=pl.BlockSpec((BN, EMBED_DIM), lambda i: (i, 0)),
        out_shape=jax.ShapeDtypeStruct((NUM_NODES, EMBED_DIM), jnp.float32),
    )(memory, w1_mem)

    pad = jnp.zeros((LIDX - 2 * EC,), jnp.int32)
    nblk = EC // BE
    chunks = []
    for c in range(NCHUNK):
        idx = jnp.concatenate(
            [lax.slice(src_nodes, (c * EC,), ((c + 1) * EC,)),
             lax.slice(dst_nodes, (c * EC,), ((c + 1) * EC,)),
             pad])
        gathered = _sc_gather(memory_proj, idx)
        g2 = gathered.reshape(BPAD // 2, 2 * EMBED_DIM)
        logits_c = pl.pallas_call(
            _edge_body,
            grid=(nblk,),
            in_specs=[
                pl.BlockSpec((BE, NODE_DIM),
                             lambda i, c=c: (i + c * nblk, 0)),
                pl.BlockSpec((BE, NODE_DIM),
                             lambda i, c=c: (i + c * nblk, 0)),
                pl.BlockSpec((BE // 2, 2 * EMBED_DIM), lambda i: (i, 0)),
                pl.BlockSpec((BE // 2, 2 * EMBED_DIM),
                             lambda i: (i + nblk, 0)),
                pl.BlockSpec((NODE_DIM, EMBED_DIM), lambda i: (0, 0)),
                pl.BlockSpec((1, EMBED_DIM), lambda i: (0, 0)),
                pl.BlockSpec((EMBED_DIM, EMBED_DIM), lambda i: (0, 0)),
                pl.BlockSpec((1, EMBED_DIM), lambda i: (0, 0)),
            ],
            out_specs=pl.BlockSpec((1, 1, BE), lambda i: (i, 0, 0)),
            out_shape=jax.ShapeDtypeStruct((nblk, 1, BE), jnp.float32),
        )(src_features, dst_features, g2, g2, w1_feat, b1r, W2, b2r)
        chunks.append(logits_c)
    return jnp.concatenate(chunks).reshape(E)
